# baseline (speedup 1.0000x reference)
import functools

import jax
import jax.numpy as jnp
from jax.experimental import pallas as pl
from jax.experimental.pallas import tpu as pltpu

_PLAN = ((64, 3, 1), (128, 4, 2), (256, 6, 2), (512, 3, 2))


def _ceil_to(x, m):
    return ((x + m - 1) // m) * m


def _tile_m(m):
    for tm in (512, 448, 392, 384, 320, 256, 224, 192, 128, 104, 88, 64, 48,
               32, 16, 8):
        if m % tm == 0:
            return tm
    return 512


def _mm_kernel(a_ref, b_ref, s_ref, t_ref, *rest, relu, has_res):
    if has_res:
        r_ref, o_ref = rest
    else:
        (o_ref,) = rest
    acc = jnp.dot(a_ref[...], b_ref[...], preferred_element_type=jnp.float32)
    out = acc * s_ref[...] + t_ref[...]
    if has_res:
        out = out + r_ref[...].astype(jnp.float32)
    if relu:
        out = jnp.maximum(out, 0.0)
    o_ref[...] = out[:, :o_ref.shape[1]].astype(o_ref.dtype)


@functools.partial(jax.jit, static_argnames=("relu", "cout"))
def _matmul_bn(a, w, scale, bias, residual=None, *, relu=True, cout=None):
    m, k = a.shape
    kp, np_ = w.shape
    if kp != k:
        a = jnp.pad(a, ((0, 0), (0, kp - k)))
    tm = _tile_m(m)
    mp = _ceil_to(m, tm)
    if mp != m:
        a = jnp.pad(a, ((0, mp - m), (0, 0)))
    tn = 256 if np_ % 256 == 0 else np_
    kk = a.shape[1]

    has_res = residual is not None
    inputs = [a, w, scale, bias]
    in_specs = [
        pl.BlockSpec((tm, kk), lambda i, j: (i, 0)),
        pl.BlockSpec((kk, tn), lambda i, j: (0, j)),
        pl.BlockSpec((1, tn), lambda i, j: (0, j)),
        pl.BlockSpec((1, tn), lambda i, j: (0, j)),
    ]
    if has_res:
        res = residual.astype(jnp.bfloat16)
        if res.shape[0] != mp:
            res = jnp.pad(res, ((0, mp - res.shape[0]), (0, 0)))
        inputs.append(res)
        in_specs.append(pl.BlockSpec((tm, tn), lambda i, j: (i, j)))

    no = np_ if (cout is None or np_ // tn > 1) else cout
    return pl.pallas_call(
        functools.partial(_mm_kernel, relu=relu, has_res=has_res),
        out_shape=jax.ShapeDtypeStruct((mp, no), jnp.bfloat16),
        grid=(mp // tm, np_ // tn),
        in_specs=in_specs,
        out_specs=pl.BlockSpec((tm, min(tn, no)), lambda i, j: (i, j)),
        compiler_params=pltpu.CompilerParams(
            dimension_semantics=("parallel", "parallel")),
    )(*inputs)


@functools.partial(jax.jit, static_argnames=("cout", "stride", "relu"))
def _conv1x1(x, p, residual=None, *, cout, stride=1, relu=True):
    if stride > 1:
        x = x[:, ::stride, ::stride, :]
    n, h, w, _ = x.shape
    a = x.reshape(n * h * w, -1)
    res = None if residual is None else residual.reshape(n * h * w, -1)
    out = _matmul_bn(a, p["w"], p["scale"], p["bias"], res, relu=relu,
                     cout=cout)
    return out[:n * h * w, :cout].reshape(n, h, w, cout)


@functools.partial(jax.jit,
                   static_argnames=("cout", "kh", "kw", "stride", "pad", "relu"))
def _conv_im2col(x, p, *, cout, kh, kw, stride, pad, relu):
    n, h, w, c = x.shape
    xp = jnp.pad(x, ((0, 0), (pad, pad), (pad, pad), (0, 0)))
    hp, wp = h + 2 * pad, w + 2 * pad
    ho = (hp - kh) // stride + 1
    wo = (wp - kw) // stride + 1
    cols = []
    for i in range(kh):
        for j in range(kw):
            cols.append(xp[:, i:i + stride * (ho - 1) + 1:stride,
                           j:j + stride * (wo - 1) + 1:stride, :])
    a = jnp.concatenate(cols, axis=-1).reshape(n * ho * wo, kh * kw * c)
    out = _matmul_bn(a, p["w"], p["scale"], p["bias"], relu=relu, cout=cout)
    return out[:n * ho * wo, :cout].reshape(n, ho, wo, cout)


def _c3_concat_kernel(x_ref, w_ref, s_ref, t_ref, o_ref, *, wp, tm):
    i = pl.program_id(1)
    halo = 2 * wp + 2
    base = pl.multiple_of(i * tm, 8)
    a_big = x_ref[pl.ds(base, tm + halo), :]
    taps = [a_big[dy * wp + dx:dy * wp + dx + tm, :]
            for dy in range(3) for dx in range(3)]
    a = jnp.concatenate(taps, axis=1)
    acc = jnp.dot(a, w_ref[...], preferred_element_type=jnp.float32)
    out = acc * s_ref[...] + t_ref[...]
    o_ref[...] = jnp.maximum(out[:, :o_ref.shape[1]], 0.0).astype(o_ref.dtype)


def _c3_taps_kernel(x_ref, w_ref, s_ref, t_ref, o_ref, *, wp, tm):
    i = pl.program_id(1)
    halo = 2 * wp + 2
    base = pl.multiple_of(i * tm, 8)
    a_big = x_ref[pl.ds(base, tm + halo), :]
    acc = None
    for dy in range(3):
        for dx in range(3):
            off = dy * wp + dx
            prod = jnp.dot(a_big[off:off + tm, :], w_ref[dy * 3 + dx],
                           preferred_element_type=jnp.float32)
            acc = prod if acc is None else acc + prod
    out = acc * s_ref[...] + t_ref[...]
    o_ref[...] = jnp.maximum(out[:, :o_ref.shape[1]], 0.0).astype(o_ref.dtype)


@functools.partial(jax.jit, static_argnames=("cout",))
def _conv3x3_fused(x, p, *, cout):
    n, h, w, cin = x.shape
    hp, wp = h + 2, w + 2
    m_img = hp * wp
    tm = 512 if m_img >= 512 else _ceil_to(m_img, 8)
    mp = _ceil_to(m_img, tm)
    np_ = p["w"].shape[2]
    tn = 256 if np_ % 256 == 0 else np_
    halo = 2 * wp + 2
    slab_rows = _ceil_to(mp + halo, 8)
    xp = jnp.pad(x, ((0, 0), (1, 1), (1, 1), (0, 0))).reshape(n, m_img, cin)
    slab = jnp.pad(xp, ((0, 0), (wp + 1, slab_rows - m_img - (wp + 1)), (0, 0)))

    wide = False and cin <= 128
    if wide:
        w2 = p["w"].reshape(9 * cin, np_)
        body = functools.partial(_c3_concat_kernel, wp=wp, tm=tm)
        w_spec = pl.BlockSpec((9 * cin, tn), lambda b, i, j: (0, j))
    else:
        w2 = p["w"]
        body = functools.partial(_c3_taps_kernel, wp=wp, tm=tm)
        w_spec = pl.BlockSpec((9, cin, tn), lambda b, i, j: (0, 0, j))

    no = np_ if np_ // tn > 1 else cout
    out = pl.pallas_call(
        body,
        out_shape=jax.ShapeDtypeStruct((n, mp, no), jnp.bfloat16),
        grid=(n, mp // tm, np_ // tn),
        in_specs=[
            pl.BlockSpec((None, slab_rows, cin), lambda b, i, j: (b, 0, 0)),
            w_spec,
            pl.BlockSpec((1, tn), lambda b, i, j: (0, j)),
            pl.BlockSpec((1, tn), lambda b, i, j: (0, j)),
        ],
        out_specs=pl.BlockSpec((None, tm, min(tn, no)), lambda b, i, j: (b, i, j)),
        compiler_params=pltpu.CompilerParams(
            dimension_semantics=("parallel", "parallel", "parallel")),
    )(slab, w2, p["scale"], p["bias"])
    out = out[:, :m_img, :cout].reshape(n, hp, wp, cout)
    return out[:, 1:1 + h, 1:1 + w, :]


_NEG = -1e30


def _pool_kernel(p00, p01, p10, p11, o_ref, *, ho, wo):
    ph = ((p00, p01), (p10, p11))
    acc = None
    for dy in range(3):
        for dx in range(3):
            v = ph[dy % 2][dx % 2][dy // 2:dy // 2 + ho,
                                   dx // 2:dx // 2 + wo, :]
            acc = v if acc is None else jnp.maximum(acc, v)
    o_ref[...] = acc


@jax.jit
def _maxpool_3x3_s2(x):
    n, h, w, c = x.shape
    xp = jnp.pad(x, ((0, 0), (1, 1), (1, 1), (0, 0)), constant_values=_NEG)
    ho = (h + 2 - 3) // 2 + 1
    wo = (w + 2 - 3) // 2 + 1
    phases = []
    for ry in (0, 1):
        for rx in (0, 1):
            ph = xp[:, ry::2, rx::2, :]
            phases.append(jnp.pad(
                ph, ((0, 0), (0, ho + 1 - ph.shape[1]),
                     (0, wo + 1 - ph.shape[2]), (0, 0)),
                constant_values=_NEG))
    return pl.pallas_call(
        functools.partial(_pool_kernel, ho=ho, wo=wo),
        out_shape=jax.ShapeDtypeStruct((n, ho, wo, c), x.dtype),
        grid=(n,),
        in_specs=[pl.BlockSpec((None, ho + 1, wo + 1, c),
                               lambda b: (b, 0, 0, 0))] * 4,
        out_specs=pl.BlockSpec((None, ho, wo, c), lambda b: (b, 0, 0, 0)),
        compiler_params=pltpu.CompilerParams(dimension_semantics=("parallel",)),
    )(*phases)


def _gmax_kernel(x_ref, o_ref):
    o_ref[...] = jnp.max(x_ref[...].astype(jnp.float32), axis=0, keepdims=True)


@jax.jit
def _global_max(x):
    n, h, w, c = x.shape
    out = pl.pallas_call(
        _gmax_kernel,
        out_shape=jax.ShapeDtypeStruct((n, 1, c), jnp.float32),
        grid=(n,),
        in_specs=[pl.BlockSpec((None, h * w, c), lambda b: (b, 0, 0))],
        out_specs=pl.BlockSpec((None, 1, c), lambda b: (b, 0, 0)),
        compiler_params=pltpu.CompilerParams(dimension_semantics=("parallel",)),
    )(x.reshape(n, h * w, c))
    return out.reshape(n, c)


def _bottleneck(x, blk, planes, stride):
    out = _conv1x1(x, blk["c1"], cout=planes, relu=True)
    if stride == 1:
        out = _conv3x3_fused(out, blk["c2"], cout=planes)
    else:
        out = _conv_im2col(out, blk["c2"], cout=planes, kh=3, kw=3,
                           stride=stride, pad=1, relu=True)
    if "ds" in blk:
        res = _conv1x1(x, blk["ds"], cout=planes * 4, stride=stride, relu=False)
    else:
        res = x
    return _conv1x1(out, blk["c3"], res, cout=planes * 4, relu=True)


def kernel(*args):
    it = iter(args)
    x = next(it)
    stem = {"w": next(it), "scale": next(it), "bias": next(it)}
    layers = []
    for planes, blocks, stride in _PLAN:
        stage = []
        for bi in range(blocks):
            blk = {}
            for nm in ("c1", "c2", "c3"):
                blk[nm] = {"w": next(it), "scale": next(it), "bias": next(it)}
            if bi == 0:
                blk["ds"] = {"w": next(it), "scale": next(it), "bias": next(it)}
            stage.append(blk)
        layers.append(stage)

    x = jnp.transpose(x, (0, 2, 3, 1)).astype(jnp.bfloat16)
    x = _conv_im2col(x, stem, cout=64, kh=7, kw=7, stride=2, pad=3, relu=False)
    x = _maxpool_3x3_s2(x)
    for (planes, blocks, stride), stage in zip(_PLAN, layers):
        for bi, blk in enumerate(stage):
            x = _bottleneck(x, blk, planes, stride if bi == 0 else 1)
    return _global_max(x)


# speedup vs baseline: 1.1759x; 1.1759x over previous
import functools

import jax
import jax.numpy as jnp
from jax.experimental import pallas as pl
from jax.experimental.pallas import tpu as pltpu

_PLAN = ((64, 3, 1), (128, 4, 2), (256, 6, 2), (512, 3, 2))


def _ceil_to(x, m):
    return ((x + m - 1) // m) * m


def _tile_m(m):
    for tm in (512, 448, 392, 384, 320, 256, 224, 192, 128, 104, 88, 64, 48,
               32, 16, 8):
        if m % tm == 0:
            return tm
    return 512


def _mm_kernel(a_ref, b_ref, s_ref, t_ref, *rest, relu, has_res):
    if has_res:
        r_ref, o_ref = rest
    else:
        (o_ref,) = rest
    acc = jnp.dot(a_ref[...], b_ref[...], preferred_element_type=jnp.float32)
    out = acc * s_ref[...] + t_ref[...]
    if has_res:
        out = out + r_ref[...].astype(jnp.float32)
    if relu:
        out = jnp.maximum(out, 0.0)
    o_ref[...] = out[:, :o_ref.shape[1]].astype(o_ref.dtype)


@functools.partial(jax.jit, static_argnames=("relu", "cout"))
def _matmul_bn(a, w, scale, bias, residual=None, *, relu=True, cout=None):
    m, k = a.shape
    kp, np_ = w.shape
    if kp != k:
        a = jnp.pad(a, ((0, 0), (0, kp - k)))
    tm = _tile_m(m)
    mp = _ceil_to(m, tm)
    if mp != m:
        a = jnp.pad(a, ((0, mp - m), (0, 0)))
    tn = 256 if np_ % 256 == 0 else np_
    kk = a.shape[1]

    has_res = residual is not None
    inputs = [a, w, scale, bias]
    in_specs = [
        pl.BlockSpec((tm, kk), lambda i, j: (i, 0)),
        pl.BlockSpec((kk, tn), lambda i, j: (0, j)),
        pl.BlockSpec((1, tn), lambda i, j: (0, j)),
        pl.BlockSpec((1, tn), lambda i, j: (0, j)),
    ]
    if has_res:
        res = residual.astype(jnp.bfloat16)
        if res.shape[0] != mp:
            res = jnp.pad(res, ((0, mp - res.shape[0]), (0, 0)))
        inputs.append(res)
        in_specs.append(pl.BlockSpec((tm, tn), lambda i, j: (i, j)))

    no = np_ if (cout is None or np_ // tn > 1) else cout
    return pl.pallas_call(
        functools.partial(_mm_kernel, relu=relu, has_res=has_res),
        out_shape=jax.ShapeDtypeStruct((mp, no), jnp.bfloat16),
        grid=(mp // tm, np_ // tn),
        in_specs=in_specs,
        out_specs=pl.BlockSpec((tm, min(tn, no)), lambda i, j: (i, j)),
        compiler_params=pltpu.CompilerParams(
            dimension_semantics=("parallel", "parallel")),
    )(*inputs)


@functools.partial(jax.jit, static_argnames=("cout", "stride", "relu"))
def _conv1x1(x, p, residual=None, *, cout, stride=1, relu=True):
    if stride > 1:
        x = x[:, ::stride, ::stride, :]
    n, h, w, _ = x.shape
    a = x.reshape(n * h * w, -1)
    res = None if residual is None else residual.reshape(n * h * w, -1)
    out = _matmul_bn(a, p["w"], p["scale"], p["bias"], res, relu=relu,
                     cout=cout)
    return out[:n * h * w, :cout].reshape(n, h, w, cout)


@functools.partial(jax.jit,
                   static_argnames=("cout", "kh", "kw", "stride", "pad", "relu"))
def _conv_im2col(x, p, *, cout, kh, kw, stride, pad, relu):
    n, h, w, c = x.shape
    xp = jnp.pad(x, ((0, 0), (pad, pad), (pad, pad), (0, 0)))
    hp, wp = h + 2 * pad, w + 2 * pad
    ho = (hp - kh) // stride + 1
    wo = (wp - kw) // stride + 1
    cols = []
    for i in range(kh):
        for j in range(kw):
            cols.append(xp[:, i:i + stride * (ho - 1) + 1:stride,
                           j:j + stride * (wo - 1) + 1:stride, :])
    a = jnp.concatenate(cols, axis=-1).reshape(n * ho * wo, kh * kw * c)
    out = _matmul_bn(a, p["w"], p["scale"], p["bias"], relu=relu, cout=cout)
    return out[:n * ho * wo, :cout].reshape(n, ho, wo, cout)


def _c3_concat_kernel(x_ref, w_ref, s_ref, t_ref, o_ref, *, wp, tm):
    i = pl.program_id(1)
    halo = 2 * wp + 2
    base = pl.multiple_of(i * tm, 8)
    a_big = x_ref[pl.ds(base, tm + halo), :]
    taps = [a_big[dy * wp + dx:dy * wp + dx + tm, :]
            for dy in range(3) for dx in range(3)]
    a = jnp.concatenate(taps, axis=1)
    acc = jnp.dot(a, w_ref[...], preferred_element_type=jnp.float32)
    out = acc * s_ref[...] + t_ref[...]
    o_ref[...] = jnp.maximum(out[:, :o_ref.shape[1]], 0.0).astype(o_ref.dtype)


def _c3_taps_kernel(x_ref, w_ref, s_ref, t_ref, o_ref, *, wp, tm):
    i = pl.program_id(1)
    halo = 2 * wp + 2
    base = pl.multiple_of(i * tm, 8)
    a_big = x_ref[pl.ds(base, tm + halo), :]
    acc = None
    for dy in range(3):
        for dx in range(3):
            off = dy * wp + dx
            prod = jnp.dot(a_big[off:off + tm, :], w_ref[dy * 3 + dx],
                           preferred_element_type=jnp.float32)
            acc = prod if acc is None else acc + prod
    out = acc * s_ref[...] + t_ref[...]
    o_ref[...] = jnp.maximum(out[:, :o_ref.shape[1]], 0.0).astype(o_ref.dtype)


@functools.partial(jax.jit, static_argnames=("cout",))
def _conv3x3_fused(x, p, *, cout):
    n, h, w, cin = x.shape
    hp, wp = h + 2, w + 2
    m_img = hp * wp
    tm = 512 if m_img >= 512 else _ceil_to(m_img, 8)
    mp = _ceil_to(m_img, tm)
    np_ = p["w"].shape[2]
    tn = 256 if np_ % 256 == 0 else np_
    halo = 2 * wp + 2
    slab_rows = _ceil_to(mp + halo, 8)
    xp = jnp.pad(x, ((0, 0), (1, 1), (1, 1), (0, 0))).reshape(n, m_img, cin)
    slab = jnp.pad(xp, ((0, 0), (wp + 1, slab_rows - m_img - (wp + 1)), (0, 0)))

    wide = False and cin <= 128
    if wide:
        w2 = p["w"].reshape(9 * cin, np_)
        body = functools.partial(_c3_concat_kernel, wp=wp, tm=tm)
        w_spec = pl.BlockSpec((9 * cin, tn), lambda b, i, j: (0, j))
    else:
        w2 = p["w"]
        body = functools.partial(_c3_taps_kernel, wp=wp, tm=tm)
        w_spec = pl.BlockSpec((9, cin, tn), lambda b, i, j: (0, 0, j))

    no = np_ if np_ // tn > 1 else cout
    out = pl.pallas_call(
        body,
        out_shape=jax.ShapeDtypeStruct((n, mp, no), jnp.bfloat16),
        grid=(n, mp // tm, np_ // tn),
        in_specs=[
            pl.BlockSpec((None, slab_rows, cin), lambda b, i, j: (b, 0, 0)),
            w_spec,
            pl.BlockSpec((1, tn), lambda b, i, j: (0, j)),
            pl.BlockSpec((1, tn), lambda b, i, j: (0, j)),
        ],
        out_specs=pl.BlockSpec((None, tm, min(tn, no)), lambda b, i, j: (b, i, j)),
        compiler_params=pltpu.CompilerParams(
            dimension_semantics=("parallel", "parallel", "parallel")),
    )(slab, w2, p["scale"], p["bias"])
    out = out[:, :m_img, :cout].reshape(n, hp, wp, cout)
    return out[:, 1:1 + h, 1:1 + w, :]


_NEG = -1e30


def _pool_kernel(x_ref, o_ref, *, h, w, c):
    v = x_ref[0]
    vp = jnp.pad(v, ((1, 1), (1, 1), (0, 0)), constant_values=_NEG)
    hp, wp = h + 2, w + 2
    p = vp.reshape(hp, wp // 2, 2, c)
    ho, wo = h // 2, w // 2
    colmax = jnp.maximum(jnp.maximum(p[:, :wo, 0], p[:, :wo, 1]),
                         p[:, 1:wo + 1, 0])
    e = colmax.reshape(hp // 2, 2, wo, c)
    even, odd = e[:, 0], e[:, 1]
    out = jnp.maximum(jnp.maximum(even[:ho], odd[:ho]), even[1:ho + 1])
    o_ref[...] = out[None]


@jax.jit
def _maxpool_3x3_s2(x):
    n, h, w, c = x.shape
    return pl.pallas_call(
        functools.partial(_pool_kernel, h=h, w=w, c=c),
        out_shape=jax.ShapeDtypeStruct((n, h // 2, w // 2, c), x.dtype),
        grid=(n,),
        in_specs=[pl.BlockSpec((1, h, w, c), lambda b: (b, 0, 0, 0))],
        out_specs=pl.BlockSpec((1, h // 2, w // 2, c), lambda b: (b, 0, 0, 0)),
        compiler_params=pltpu.CompilerParams(dimension_semantics=("parallel",)),
    )(x)


def _gmax_kernel(x_ref, o_ref):
    o_ref[...] = jnp.max(x_ref[...].astype(jnp.float32), axis=0, keepdims=True)


@jax.jit
def _global_max(x):
    n, h, w, c = x.shape
    out = pl.pallas_call(
        _gmax_kernel,
        out_shape=jax.ShapeDtypeStruct((n, 1, c), jnp.float32),
        grid=(n,),
        in_specs=[pl.BlockSpec((None, h * w, c), lambda b: (b, 0, 0))],
        out_specs=pl.BlockSpec((None, 1, c), lambda b: (b, 0, 0)),
        compiler_params=pltpu.CompilerParams(dimension_semantics=("parallel",)),
    )(x.reshape(n, h * w, c))
    return out.reshape(n, c)


def _bottleneck(x, blk, planes, stride):
    out = _conv1x1(x, blk["c1"], cout=planes, relu=True)
    if stride == 1:
        out = _conv3x3_fused(out, blk["c2"], cout=planes)
    else:
        out = _conv_im2col(out, blk["c2"], cout=planes, kh=3, kw=3,
                           stride=stride, pad=1, relu=True)
    if "ds" in blk:
        res = _conv1x1(x, blk["ds"], cout=planes * 4, stride=stride, relu=False)
    else:
        res = x
    return _conv1x1(out, blk["c3"], res, cout=planes * 4, relu=True)


def kernel(*args):
    it = iter(args)
    x = next(it)
    stem = {"w": next(it), "scale": next(it), "bias": next(it)}
    layers = []
    for planes, blocks, stride in _PLAN:
        stage = []
        for bi in range(blocks):
            blk = {}
            for nm in ("c1", "c2", "c3"):
                blk[nm] = {"w": next(it), "scale": next(it), "bias": next(it)}
            if bi == 0:
                blk["ds"] = {"w": next(it), "scale": next(it), "bias": next(it)}
            stage.append(blk)
        layers.append(stage)

    x = jnp.transpose(x, (0, 2, 3, 1)).astype(jnp.bfloat16)
    x = _conv_im2col(x, stem, cout=64, kh=7, kw=7, stride=2, pad=3, relu=False)
    x = _maxpool_3x3_s2(x)
    for (planes, blocks, stride), stage in zip(_PLAN, layers):
        for bi, blk in enumerate(stage):
            x = _bottleneck(x, blk, planes, stride if bi == 0 else 1)
    return _global_max(x)


# speedup vs baseline: 1.3358x; 1.1360x over previous
import functools

import jax
import jax.numpy as jnp
from jax.experimental import pallas as pl
from jax.experimental.pallas import tpu as pltpu

_PLAN = ((64, 3, 1), (128, 4, 2), (256, 6, 2), (512, 3, 2))


def _ceil_to(x, m):
    return ((x + m - 1) // m) * m


def _tile_m(m):
    for tm in (512, 448, 392, 384, 320, 256, 224, 192, 128, 104, 88, 64, 48,
               32, 16, 8):
        if m % tm == 0:
            return tm
    return 512


def _mm_kernel(a_ref, b_ref, s_ref, t_ref, *rest, relu, has_res):
    if has_res:
        r_ref, o_ref = rest
    else:
        (o_ref,) = rest
    acc = jnp.dot(a_ref[...], b_ref[...], preferred_element_type=jnp.float32)
    out = acc * s_ref[...] + t_ref[...]
    if has_res:
        out = out + r_ref[...].astype(jnp.float32)
    if relu:
        out = jnp.maximum(out, 0.0)
    o_ref[...] = out[:, :o_ref.shape[1]].astype(o_ref.dtype)


@functools.partial(jax.jit, static_argnames=("relu", "cout"))
def _matmul_bn(a, w, scale, bias, residual=None, *, relu=True, cout=None):
    m, k = a.shape
    kp, np_ = w.shape
    if kp != k:
        a = jnp.pad(a, ((0, 0), (0, kp - k)))
    tm = _tile_m(m)
    mp = _ceil_to(m, tm)
    if mp != m:
        a = jnp.pad(a, ((0, mp - m), (0, 0)))
    tn = 256 if np_ % 256 == 0 else np_
    kk = a.shape[1]

    has_res = residual is not None
    inputs = [a, w, scale, bias]
    in_specs = [
        pl.BlockSpec((tm, kk), lambda i, j: (i, 0)),
        pl.BlockSpec((kk, tn), lambda i, j: (0, j)),
        pl.BlockSpec((1, tn), lambda i, j: (0, j)),
        pl.BlockSpec((1, tn), lambda i, j: (0, j)),
    ]
    if has_res:
        res = residual.astype(jnp.bfloat16)
        if res.shape[0] != mp:
            res = jnp.pad(res, ((0, mp - res.shape[0]), (0, 0)))
        inputs.append(res)
        in_specs.append(pl.BlockSpec((tm, tn), lambda i, j: (i, j)))

    no = np_ if (cout is None or np_ // tn > 1) else cout
    return pl.pallas_call(
        functools.partial(_mm_kernel, relu=relu, has_res=has_res),
        out_shape=jax.ShapeDtypeStruct((mp, no), jnp.bfloat16),
        grid=(mp // tm, np_ // tn),
        in_specs=in_specs,
        out_specs=pl.BlockSpec((tm, min(tn, no)), lambda i, j: (i, j)),
        compiler_params=pltpu.CompilerParams(
            dimension_semantics=("parallel", "parallel")),
    )(*inputs)


@functools.partial(jax.jit, static_argnames=("cout", "stride", "relu"))
def _conv1x1(x, p, residual=None, *, cout, stride=1, relu=True):
    if stride > 1:
        x = x[:, ::stride, ::stride, :]
    n, h, w, _ = x.shape
    a = x.reshape(n * h * w, -1)
    res = None if residual is None else residual.reshape(n * h * w, -1)
    out = _matmul_bn(a, p["w"], p["scale"], p["bias"], res, relu=relu,
                     cout=cout)
    return out[:n * h * w, :cout].reshape(n, h, w, cout)


@functools.partial(jax.jit,
                   static_argnames=("cout", "kh", "kw", "stride", "pad", "relu"))
def _conv_im2col(x, p, *, cout, kh, kw, stride, pad, relu):
    n, h, w, c = x.shape
    xp = jnp.pad(x, ((0, 0), (pad, pad), (pad, pad), (0, 0)))
    hp, wp = h + 2 * pad, w + 2 * pad
    ho = (hp - kh) // stride + 1
    wo = (wp - kw) // stride + 1
    cols = []
    for i in range(kh):
        for j in range(kw):
            cols.append(xp[:, i:i + stride * (ho - 1) + 1:stride,
                           j:j + stride * (wo - 1) + 1:stride, :])
    a = jnp.concatenate(cols, axis=-1).reshape(n * ho * wo, kh * kw * c)
    out = _matmul_bn(a, p["w"], p["scale"], p["bias"], relu=relu, cout=cout)
    return out[:n * ho * wo, :cout].reshape(n, ho, wo, cout)


def _c3_concat_kernel(x_ref, w_ref, s_ref, t_ref, o_ref, *, wp, tm):
    i = pl.program_id(1)
    halo = 2 * wp + 2
    base = pl.multiple_of(i * tm, 8)
    a_big = x_ref[pl.ds(base, tm + halo), :]
    taps = [a_big[dy * wp + dx:dy * wp + dx + tm, :]
            for dy in range(3) for dx in range(3)]
    a = jnp.concatenate(taps, axis=1)
    acc = jnp.dot(a, w_ref[...], preferred_element_type=jnp.float32)
    out = acc * s_ref[...] + t_ref[...]
    o_ref[...] = jnp.maximum(out[:, :o_ref.shape[1]], 0.0).astype(o_ref.dtype)


def _c3_taps_kernel(x_ref, w_ref, s_ref, t_ref, o_ref, *, wp, tm):
    i = pl.program_id(1)
    halo = 2 * wp + 2
    base = pl.multiple_of(i * tm, 8)
    a_big = x_ref[pl.ds(base, tm + halo), :]
    acc = None
    for dy in range(3):
        for dx in range(3):
            off = dy * wp + dx
            prod = jnp.dot(a_big[off:off + tm, :], w_ref[dy * 3 + dx],
                           preferred_element_type=jnp.float32)
            acc = prod if acc is None else acc + prod
    out = acc * s_ref[...] + t_ref[...]
    o_ref[...] = jnp.maximum(out[:, :o_ref.shape[1]], 0.0).astype(o_ref.dtype)


@functools.partial(jax.jit, static_argnames=("cout",))
def _conv3x3_fused(x, p, *, cout):
    n, h, w, cin = x.shape
    hp, wp = h + 2, w + 2
    m_img = hp * wp
    tm = 512 if m_img >= 512 else _ceil_to(m_img, 8)
    mp = _ceil_to(m_img, tm)
    np_ = p["w"].shape[2]
    tn = 256 if np_ % 256 == 0 else np_
    halo = 2 * wp + 2
    slab_rows = _ceil_to(mp + halo, 8)
    xp = jnp.pad(x, ((0, 0), (1, 1), (1, 1), (0, 0))).reshape(n, m_img, cin)
    slab = jnp.pad(xp, ((0, 0), (wp + 1, slab_rows - m_img - (wp + 1)), (0, 0)))

    wide = False and cin <= 128
    if wide:
        w2 = p["w"].reshape(9 * cin, np_)
        body = functools.partial(_c3_concat_kernel, wp=wp, tm=tm)
        w_spec = pl.BlockSpec((9 * cin, tn), lambda b, i, j: (0, j))
    else:
        w2 = p["w"]
        body = functools.partial(_c3_taps_kernel, wp=wp, tm=tm)
        w_spec = pl.BlockSpec((9, cin, tn), lambda b, i, j: (0, 0, j))

    no = np_ if np_ // tn > 1 else cout
    out = pl.pallas_call(
        body,
        out_shape=jax.ShapeDtypeStruct((n, mp, no), jnp.bfloat16),
        grid=(n, mp // tm, np_ // tn),
        in_specs=[
            pl.BlockSpec((None, slab_rows, cin), lambda b, i, j: (b, 0, 0)),
            w_spec,
            pl.BlockSpec((1, tn), lambda b, i, j: (0, j)),
            pl.BlockSpec((1, tn), lambda b, i, j: (0, j)),
        ],
        out_specs=pl.BlockSpec((None, tm, min(tn, no)), lambda b, i, j: (b, i, j)),
        compiler_params=pltpu.CompilerParams(
            dimension_semantics=("parallel", "parallel", "parallel")),
    )(slab, w2, p["scale"], p["bias"])
    out = out[:, :m_img, :cout].reshape(n, hp, wp, cout)
    return out[:, 1:1 + h, 1:1 + w, :]


def _stem_kernel(x_ref, w_ref, s_ref, t_ref, o_ref, *, wp, tm):
    i = pl.program_id(1)
    base = pl.multiple_of(i * tm, 8)
    a_big = x_ref[pl.ds(base, tm + 3 * wp + 3), :]
    acc = None
    for a in range(4):
        for c in range(4):
            off = a * wp + c
            prod = jnp.dot(a_big[off:off + tm, :], w_ref[a * 4 + c],
                           preferred_element_type=jnp.float32)
            acc = prod if acc is None else acc + prod
    out = acc * s_ref[...] + t_ref[...]
    o_ref[...] = out[:, :o_ref.shape[1]].astype(o_ref.dtype)


@jax.jit
def _stem_conv(x_nchw, p):
    n = x_nchw.shape[0]
    x = jnp.transpose(x_nchw, (0, 2, 3, 1)).astype(jnp.bfloat16)
    xp = jnp.pad(x, ((0, 0), (3, 3), (3, 3), (0, 0)))
    ph = xp.reshape(n, 115, 2, 115, 2, 3).transpose(0, 1, 3, 2, 4, 5)
    ph = ph.reshape(n, 115 * 115, 12)
    wp2 = 115
    m_img = 115 * 115
    tm = 512
    mp = _ceil_to(m_img, tm)
    slab_rows = _ceil_to(mp + 3 * wp2 + 3, 8)
    slab = jnp.pad(ph, ((0, 0), (0, slab_rows - m_img), (0, 0)))

    np_ = p["w"].shape[1]
    idx = []
    for a in range(4):
        for c in range(4):
            for b in range(2):
                for d in range(2):
                    for ch in range(3):
                        ky, kx = 2 * a + b, 2 * c + d
                        idx.append(ky * 21 + kx * 3 + ch if ky < 7 and kx < 7
                                   else 147)
    w4 = jnp.take(p["w"], jnp.asarray(idx), axis=0).reshape(16, 12, np_)

    out = pl.pallas_call(
        functools.partial(_stem_kernel, wp=wp2, tm=tm),
        out_shape=jax.ShapeDtypeStruct((n, mp, 64), jnp.bfloat16),
        grid=(n, mp // tm, 1),
        in_specs=[
            pl.BlockSpec((None, slab_rows, 12), lambda b, i, j: (b, 0, 0)),
            pl.BlockSpec((16, 12, np_), lambda b, i, j: (0, 0, 0)),
            pl.BlockSpec((1, np_), lambda b, i, j: (0, 0)),
            pl.BlockSpec((1, np_), lambda b, i, j: (0, 0)),
        ],
        out_specs=pl.BlockSpec((None, tm, 64), lambda b, i, j: (b, i, 0)),
        compiler_params=pltpu.CompilerParams(
            dimension_semantics=("parallel", "parallel", "parallel")),
    )(slab, w4, p["scale"], p["bias"])
    return out[:, :m_img, :].reshape(n, 115, 115, 64)


_NEG = -1e30


def _pool_kernel(x_ref, o_ref, *, h, w, c):
    v = x_ref[0]
    vp = jnp.pad(v, ((1, 1), (1, 1), (0, 0)), constant_values=_NEG)
    hp, wp = h + 2, w + 2
    p = vp.reshape(hp, wp // 2, 2, c)
    ho, wo = h // 2, w // 2
    colmax = jnp.maximum(jnp.maximum(p[:, :wo, 0], p[:, :wo, 1]),
                         p[:, 1:wo + 1, 0])
    e = colmax.reshape(hp // 2, 2, wo, c)
    even, odd = e[:, 0], e[:, 1]
    out = jnp.maximum(jnp.maximum(even[:ho], odd[:ho]), even[1:ho + 1])
    o_ref[...] = out[None]


@jax.jit
def _maxpool_3x3_s2(x):
    n, h, w, c = x.shape
    return pl.pallas_call(
        functools.partial(_pool_kernel, h=h, w=w, c=c),
        out_shape=jax.ShapeDtypeStruct((n, h // 2, w // 2, c), x.dtype),
        grid=(n,),
        in_specs=[pl.BlockSpec((1, h, w, c), lambda b: (b, 0, 0, 0))],
        out_specs=pl.BlockSpec((1, h // 2, w // 2, c), lambda b: (b, 0, 0, 0)),
        compiler_params=pltpu.CompilerParams(dimension_semantics=("parallel",)),
    )(x)


def _gmax_kernel(x_ref, o_ref):
    o_ref[...] = jnp.max(x_ref[...].astype(jnp.float32), axis=0, keepdims=True)


@jax.jit
def _global_max(x):
    n, h, w, c = x.shape
    out = pl.pallas_call(
        _gmax_kernel,
        out_shape=jax.ShapeDtypeStruct((n, 1, c), jnp.float32),
        grid=(n,),
        in_specs=[pl.BlockSpec((None, h * w, c), lambda b: (b, 0, 0))],
        out_specs=pl.BlockSpec((None, 1, c), lambda b: (b, 0, 0)),
        compiler_params=pltpu.CompilerParams(dimension_semantics=("parallel",)),
    )(x.reshape(n, h * w, c))
    return out.reshape(n, c)


def _bottleneck(x, blk, planes, stride):
    out = _conv1x1(x, blk["c1"], cout=planes, relu=True)
    if stride == 1:
        out = _conv3x3_fused(out, blk["c2"], cout=planes)
    else:
        out = _conv_im2col(out, blk["c2"], cout=planes, kh=3, kw=3,
                           stride=stride, pad=1, relu=True)
    if "ds" in blk:
        res = _conv1x1(x, blk["ds"], cout=planes * 4, stride=stride, relu=False)
    else:
        res = x
    return _conv1x1(out, blk["c3"], res, cout=planes * 4, relu=True)


def kernel(*args):
    it = iter(args)
    x = next(it)
    stem = {"w": next(it), "scale": next(it), "bias": next(it)}
    layers = []
    for planes, blocks, stride in _PLAN:
        stage = []
        for bi in range(blocks):
            blk = {}
            for nm in ("c1", "c2", "c3"):
                blk[nm] = {"w": next(it), "scale": next(it), "bias": next(it)}
            if bi == 0:
                blk["ds"] = {"w": next(it), "scale": next(it), "bias": next(it)}
            stage.append(blk)
        layers.append(stage)

    x = _stem_conv(x, stem)[:, :112, :112, :]
    x = _maxpool_3x3_s2(x)
    for (planes, blocks, stride), stage in zip(_PLAN, layers):
        for bi, blk in enumerate(stage):
            x = _bottleneck(x, blk, planes, stride if bi == 0 else 1)
    return _global_max(x)


# speedup vs baseline: 2.1186x; 1.5860x over previous
import functools

import jax
import jax.numpy as jnp
from jax.experimental import pallas as pl
from jax.experimental.pallas import tpu as pltpu

_PLAN = ((64, 3, 1), (128, 4, 2), (256, 6, 2), (512, 3, 2))


def _ceil_to(x, m):
    return ((x + m - 1) // m) * m


def _tile_m(m):
    for tm in (512, 448, 392, 384, 320, 256, 224, 192, 128, 104, 88, 64, 48,
               32, 16, 8):
        if m % tm == 0:
            return tm
    return 512


def _mm_kernel(a_ref, b_ref, s_ref, t_ref, *rest, relu, has_res):
    if has_res:
        r_ref, o_ref = rest
    else:
        (o_ref,) = rest
    acc = jnp.dot(a_ref[...], b_ref[...], preferred_element_type=jnp.float32)
    out = acc * s_ref[...] + t_ref[...]
    if has_res:
        out = out + r_ref[...].astype(jnp.float32)
    if relu:
        out = jnp.maximum(out, 0.0)
    o_ref[...] = out[:, :o_ref.shape[1]].astype(o_ref.dtype)


@functools.partial(jax.jit, static_argnames=("relu", "cout"))
def _matmul_bn(a, w, scale, bias, residual=None, *, relu=True, cout=None):
    m, k = a.shape
    kp, np_ = w.shape
    if kp != k:
        a = jnp.pad(a, ((0, 0), (0, kp - k)))
    tm = _tile_m(m)
    mp = _ceil_to(m, tm)
    if mp != m:
        a = jnp.pad(a, ((0, mp - m), (0, 0)))
    tn = 256 if np_ % 256 == 0 else np_
    kk = a.shape[1]

    has_res = residual is not None
    inputs = [a, w, scale, bias]
    in_specs = [
        pl.BlockSpec((tm, kk), lambda i, j: (i, 0)),
        pl.BlockSpec((kk, tn), lambda i, j: (0, j)),
        pl.BlockSpec((1, tn), lambda i, j: (0, j)),
        pl.BlockSpec((1, tn), lambda i, j: (0, j)),
    ]
    if has_res:
        res = residual.astype(jnp.bfloat16)
        if res.shape[0] != mp:
            res = jnp.pad(res, ((0, mp - res.shape[0]), (0, 0)))
        inputs.append(res)
        in_specs.append(pl.BlockSpec((tm, tn), lambda i, j: (i, j)))

    no = np_ if (cout is None or np_ // tn > 1) else cout
    return pl.pallas_call(
        functools.partial(_mm_kernel, relu=relu, has_res=has_res),
        out_shape=jax.ShapeDtypeStruct((mp, no), jnp.bfloat16),
        grid=(mp // tm, np_ // tn),
        in_specs=in_specs,
        out_specs=pl.BlockSpec((tm, min(tn, no)), lambda i, j: (i, j)),
        compiler_params=pltpu.CompilerParams(
            dimension_semantics=("parallel", "parallel")),
    )(*inputs)


@functools.partial(jax.jit, static_argnames=("cout", "stride", "relu"))
def _conv1x1(x, p, residual=None, *, cout, stride=1, relu=True):
    if stride > 1:
        x = x[:, ::stride, ::stride, :]
    n, h, w, _ = x.shape
    a = x.reshape(n * h * w, -1)
    res = None if residual is None else residual.reshape(n * h * w, -1)
    out = _matmul_bn(a, p["w"], p["scale"], p["bias"], res, relu=relu,
                     cout=cout)
    return out[:n * h * w, :cout].reshape(n, h, w, cout)


@functools.partial(jax.jit,
                   static_argnames=("cout", "kh", "kw", "stride", "pad", "relu"))
def _conv_im2col(x, p, *, cout, kh, kw, stride, pad, relu):
    n, h, w, c = x.shape
    xp = jnp.pad(x, ((0, 0), (pad, pad), (pad, pad), (0, 0)))
    hp, wp = h + 2 * pad, w + 2 * pad
    ho = (hp - kh) // stride + 1
    wo = (wp - kw) // stride + 1
    cols = []
    for i in range(kh):
        for j in range(kw):
            cols.append(xp[:, i:i + stride * (ho - 1) + 1:stride,
                           j:j + stride * (wo - 1) + 1:stride, :])
    a = jnp.concatenate(cols, axis=-1).reshape(n * ho * wo, kh * kw * c)
    out = _matmul_bn(a, p["w"], p["scale"], p["bias"], relu=relu, cout=cout)
    return out[:n * ho * wo, :cout].reshape(n, ho, wo, cout)


def _c3_concat_kernel(x_ref, w_ref, s_ref, t_ref, o_ref, *, wp, tm):
    i = pl.program_id(1)
    halo = 2 * wp + 2
    base = pl.multiple_of(i * tm, 8)
    a_big = x_ref[pl.ds(base, tm + halo), :]
    taps = [a_big[dy * wp + dx:dy * wp + dx + tm, :]
            for dy in range(3) for dx in range(3)]
    a = jnp.concatenate(taps, axis=1)
    acc = jnp.dot(a, w_ref[...], preferred_element_type=jnp.float32)
    out = acc * s_ref[...] + t_ref[...]
    o_ref[...] = jnp.maximum(out[:, :o_ref.shape[1]], 0.0).astype(o_ref.dtype)


def _c3_taps_kernel(x_ref, w_ref, s_ref, t_ref, o_ref, *, wp, tm):
    i = pl.program_id(1)
    halo = 2 * wp + 2
    base = pl.multiple_of(i * tm, 8)
    a_big = x_ref[pl.ds(base, tm + halo), :]
    acc = None
    for dy in range(3):
        for dx in range(3):
            off = dy * wp + dx
            prod = jnp.dot(a_big[off:off + tm, :], w_ref[dy * 3 + dx],
                           preferred_element_type=jnp.float32)
            acc = prod if acc is None else acc + prod
    out = acc * s_ref[...] + t_ref[...]
    o_ref[...] = jnp.maximum(out[:, :o_ref.shape[1]], 0.0).astype(o_ref.dtype)


@functools.partial(jax.jit, static_argnames=("cout",))
def _conv3x3_fused(x, p, *, cout):
    n, h, w, cin = x.shape
    hp, wp = h + 2, w + 2
    m_img = hp * wp
    tm = 512 if m_img >= 512 else _ceil_to(m_img, 8)
    mp = _ceil_to(m_img, tm)
    np_ = p["w"].shape[2]
    tn = 256 if np_ % 256 == 0 else np_
    halo = 2 * wp + 2
    slab_rows = _ceil_to(mp + halo, 8)
    xp = jnp.pad(x, ((0, 0), (1, 1), (1, 1), (0, 0))).reshape(n, m_img, cin)
    slab = jnp.pad(xp, ((0, 0), (wp + 1, slab_rows - m_img - (wp + 1)), (0, 0)))

    wide = False and cin <= 128
    if wide:
        w2 = p["w"].reshape(9 * cin, np_)
        body = functools.partial(_c3_concat_kernel, wp=wp, tm=tm)
        w_spec = pl.BlockSpec((9 * cin, tn), lambda b, i, j: (0, j))
    else:
        w2 = p["w"]
        body = functools.partial(_c3_taps_kernel, wp=wp, tm=tm)
        w_spec = pl.BlockSpec((9, cin, tn), lambda b, i, j: (0, 0, j))

    no = np_ if np_ // tn > 1 else cout
    out = pl.pallas_call(
        body,
        out_shape=jax.ShapeDtypeStruct((n, mp, no), jnp.bfloat16),
        grid=(n, mp // tm, np_ // tn),
        in_specs=[
            pl.BlockSpec((None, slab_rows, cin), lambda b, i, j: (b, 0, 0)),
            w_spec,
            pl.BlockSpec((1, tn), lambda b, i, j: (0, j)),
            pl.BlockSpec((1, tn), lambda b, i, j: (0, j)),
        ],
        out_specs=pl.BlockSpec((None, tm, min(tn, no)), lambda b, i, j: (b, i, j)),
        compiler_params=pltpu.CompilerParams(
            dimension_semantics=("parallel", "parallel", "parallel")),
    )(slab, w2, p["scale"], p["bias"])
    out = out[:, :m_img, :cout].reshape(n, hp, wp, cout)
    return out[:, 1:1 + h, 1:1 + w, :]


def _stem_kernel(x_ref, w_ref, s_ref, t_ref, o_ref, *, wp, tm):
    i = pl.program_id(1)
    base = pl.multiple_of(i * tm, 8)
    a_big = x_ref[pl.ds(base, tm + 3 * wp + 3), :]
    acc = None
    for a in range(4):
        for c in range(4):
            off = a * wp + c
            prod = jnp.dot(a_big[off:off + tm, :], w_ref[a * 4 + c],
                           preferred_element_type=jnp.float32)
            acc = prod if acc is None else acc + prod
    out = acc * s_ref[...] + t_ref[...]
    o_ref[...] = out[:, :o_ref.shape[1]].astype(o_ref.dtype)


@jax.jit
def _stem_conv(x_nchw, p):
    n = x_nchw.shape[0]
    x = jnp.transpose(x_nchw, (0, 2, 3, 1)).astype(jnp.bfloat16)
    xp = jnp.pad(x, ((0, 0), (3, 3), (3, 3), (0, 0)))
    ph = xp.reshape(n, 115, 2, 115, 2, 3).transpose(0, 1, 3, 2, 4, 5)
    ph = ph.reshape(n, 115 * 115, 12)
    wp2 = 115
    m_img = 115 * 115
    tm = 512
    mp = _ceil_to(m_img, tm)
    slab_rows = _ceil_to(mp + 3 * wp2 + 3, 8)
    slab = jnp.pad(ph, ((0, 0), (0, slab_rows - m_img), (0, 0)))

    np_ = p["w"].shape[1]
    idx = []
    for a in range(4):
        for c in range(4):
            for b in range(2):
                for d in range(2):
                    for ch in range(3):
                        ky, kx = 2 * a + b, 2 * c + d
                        idx.append(ky * 21 + kx * 3 + ch if ky < 7 and kx < 7
                                   else 147)
    w4 = jnp.take(p["w"], jnp.asarray(idx), axis=0).reshape(16, 12, np_)

    out = pl.pallas_call(
        functools.partial(_stem_kernel, wp=wp2, tm=tm),
        out_shape=jax.ShapeDtypeStruct((n, mp, 64), jnp.bfloat16),
        grid=(n, mp // tm, 1),
        in_specs=[
            pl.BlockSpec((None, slab_rows, 12), lambda b, i, j: (b, 0, 0)),
            pl.BlockSpec((16, 12, np_), lambda b, i, j: (0, 0, 0)),
            pl.BlockSpec((1, np_), lambda b, i, j: (0, 0)),
            pl.BlockSpec((1, np_), lambda b, i, j: (0, 0)),
        ],
        out_specs=pl.BlockSpec((None, tm, 64), lambda b, i, j: (b, i, 0)),
        compiler_params=pltpu.CompilerParams(
            dimension_semantics=("parallel", "parallel", "parallel")),
    )(slab, w4, p["scale"], p["bias"])
    return out[:, :m_img, :].reshape(n, 115, 115, 64)


def _s2_kernel(x_ref, w_ref, s_ref, t_ref, o_ref, *, wp):
    acc = None
    for a in range(2):
        for c in range(2):
            off = a * wp + c
            prod = jnp.dot(x_ref[off:off + o_ref.shape[0], :], w_ref[a * 2 + c],
                           preferred_element_type=jnp.float32)
            acc = prod if acc is None else acc + prod
    out = acc * s_ref[...] + t_ref[...]
    o_ref[...] = jnp.maximum(out, 0.0).astype(o_ref.dtype)


@functools.partial(jax.jit, static_argnames=("cout",))
def _conv3x3_s2(x, p, *, cout):
    n, h, w, c = x.shape
    h2, w2 = (h + 2) // 2, (w + 2) // 2
    xp = jnp.pad(x, ((0, 0), (1, 1), (1, 1), (0, 0)))
    ph = xp.reshape(n, h2, 2, w2, 2, c).transpose(0, 1, 3, 2, 4, 5)
    ph = ph.reshape(n, h2 * w2, 4 * c)
    m_img = h2 * w2
    mp = _ceil_to(m_img, 8)
    slab_rows = _ceil_to(mp + w2 + 1, 8)
    slab = jnp.pad(ph, ((0, 0), (0, slab_rows - m_img), (0, 0)))

    kp, np_ = p["w"].shape
    wz = jnp.concatenate([p["w"], jnp.zeros((8, np_), p["w"].dtype)], axis=0)
    idx = []
    for a in range(2):
        for cc in range(2):
            for b in range(2):
                for d in range(2):
                    ky, kx = 2 * a + b, 2 * cc + d
                    for ch in range(c):
                        idx.append(ky * 3 * c + kx * c + ch
                                   if ky < 3 and kx < 3 else kp)
    w4 = jnp.take(wz, jnp.asarray(idx), axis=0).reshape(4, 4 * c, np_)

    tn = 256 if np_ % 256 == 0 else np_
    ho, wo = h // 2, w // 2
    out = pl.pallas_call(
        functools.partial(_s2_kernel, wp=w2),
        out_shape=jax.ShapeDtypeStruct((n, mp, np_), jnp.bfloat16),
        grid=(n, np_ // tn),
        in_specs=[
            pl.BlockSpec((None, slab_rows, 4 * c), lambda b, j: (b, 0, 0)),
            pl.BlockSpec((4, 4 * c, tn), lambda b, j: (0, 0, j)),
            pl.BlockSpec((1, tn), lambda b, j: (0, j)),
            pl.BlockSpec((1, tn), lambda b, j: (0, j)),
        ],
        out_specs=pl.BlockSpec((None, mp, tn), lambda b, j: (b, 0, j)),
        compiler_params=pltpu.CompilerParams(
            dimension_semantics=("parallel", "parallel")),
    )(slab, w4, p["scale"], p["bias"])
    out = out[:, :m_img, :cout].reshape(n, h2, w2, cout)
    return out[:, :ho, :wo, :]


_NEG = -1e30


def _pool_kernel(x_ref, o_ref, *, h, w, c):
    v = x_ref[0]
    vp = jnp.pad(v, ((1, 1), (1, 1), (0, 0)), constant_values=_NEG)
    hp, wp = h + 2, w + 2
    p = vp.reshape(hp, wp // 2, 2, c)
    ho, wo = h // 2, w // 2
    colmax = jnp.maximum(jnp.maximum(p[:, :wo, 0], p[:, :wo, 1]),
                         p[:, 1:wo + 1, 0])
    e = colmax.reshape(hp // 2, 2, wo, c)
    even, odd = e[:, 0], e[:, 1]
    out = jnp.maximum(jnp.maximum(even[:ho], odd[:ho]), even[1:ho + 1])
    o_ref[...] = out[None]


@jax.jit
def _maxpool_3x3_s2(x):
    n, h, w, c = x.shape
    return pl.pallas_call(
        functools.partial(_pool_kernel, h=h, w=w, c=c),
        out_shape=jax.ShapeDtypeStruct((n, h // 2, w // 2, c), x.dtype),
        grid=(n,),
        in_specs=[pl.BlockSpec((1, h, w, c), lambda b: (b, 0, 0, 0))],
        out_specs=pl.BlockSpec((1, h // 2, w // 2, c), lambda b: (b, 0, 0, 0)),
        compiler_params=pltpu.CompilerParams(dimension_semantics=("parallel",)),
    )(x)


def _gmax_kernel(x_ref, o_ref):
    o_ref[...] = jnp.max(x_ref[...].astype(jnp.float32), axis=0, keepdims=True)


@jax.jit
def _global_max(x):
    n, h, w, c = x.shape
    out = pl.pallas_call(
        _gmax_kernel,
        out_shape=jax.ShapeDtypeStruct((n, 1, c), jnp.float32),
        grid=(n,),
        in_specs=[pl.BlockSpec((None, h * w, c), lambda b: (b, 0, 0))],
        out_specs=pl.BlockSpec((None, 1, c), lambda b: (b, 0, 0)),
        compiler_params=pltpu.CompilerParams(dimension_semantics=("parallel",)),
    )(x.reshape(n, h * w, c))
    return out.reshape(n, c)


def _bottleneck(x, blk, planes, stride):
    out = _conv1x1(x, blk["c1"], cout=planes, relu=True)
    if stride == 1:
        out = _conv3x3_fused(out, blk["c2"], cout=planes)
    else:
        out = _conv3x3_s2(out, blk["c2"], cout=planes)
    if "ds" in blk:
        res = _conv1x1(x, blk["ds"], cout=planes * 4, stride=stride, relu=False)
    else:
        res = x
    return _conv1x1(out, blk["c3"], res, cout=planes * 4, relu=True)


def kernel(*args):
    it = iter(args)
    x = next(it)
    stem = {"w": next(it), "scale": next(it), "bias": next(it)}
    layers = []
    for planes, blocks, stride in _PLAN:
        stage = []
        for bi in range(blocks):
            blk = {}
            for nm in ("c1", "c2", "c3"):
                blk[nm] = {"w": next(it), "scale": next(it), "bias": next(it)}
            if bi == 0:
                blk["ds"] = {"w": next(it), "scale": next(it), "bias": next(it)}
            stage.append(blk)
        layers.append(stage)

    x = _stem_conv(x, stem)[:, :112, :112, :]
    x = _maxpool_3x3_s2(x)
    for (planes, blocks, stride), stage in zip(_PLAN, layers):
        for bi, blk in enumerate(stage):
            x = _bottleneck(x, blk, planes, stride if bi == 0 else 1)
    return _global_max(x)


# speedup vs baseline: 2.1228x; 1.0020x over previous
import functools

import jax
import jax.numpy as jnp
from jax.experimental import pallas as pl
from jax.experimental.pallas import tpu as pltpu

_PLAN = ((64, 3, 1), (128, 4, 2), (256, 6, 2), (512, 3, 2))


def _ceil_to(x, m):
    return ((x + m - 1) // m) * m


def _tile_m(m):
    for tm in (512, 448, 392, 384, 320, 256, 224, 192, 128, 104, 88, 64, 48,
               32, 16, 8):
        if m % tm == 0:
            return tm
    return 512


def _mm_kernel(a_ref, b_ref, s_ref, t_ref, *rest, relu, has_res):
    if has_res:
        r_ref, o_ref = rest
    else:
        (o_ref,) = rest
    acc = jnp.dot(a_ref[...], b_ref[...], preferred_element_type=jnp.float32)
    out = acc * s_ref[...] + t_ref[...]
    if has_res:
        out = out + r_ref[...].astype(jnp.float32)
    if relu:
        out = jnp.maximum(out, 0.0)
    o_ref[...] = out[:, :o_ref.shape[1]].astype(o_ref.dtype)


@functools.partial(jax.jit, static_argnames=("relu", "cout"))
def _matmul_bn(a, w, scale, bias, residual=None, *, relu=True, cout=None):
    m, k = a.shape
    kp, np_ = w.shape
    if kp != k:
        a = jnp.pad(a, ((0, 0), (0, kp - k)))
    tm = _tile_m(m)
    mp = _ceil_to(m, tm)
    if mp != m:
        a = jnp.pad(a, ((0, mp - m), (0, 0)))
    tn = 256 if np_ % 256 == 0 else np_
    kk = a.shape[1]

    has_res = residual is not None
    inputs = [a, w, scale, bias]
    in_specs = [
        pl.BlockSpec((tm, kk), lambda i, j: (i, 0)),
        pl.BlockSpec((kk, tn), lambda i, j: (0, j)),
        pl.BlockSpec((1, tn), lambda i, j: (0, j)),
        pl.BlockSpec((1, tn), lambda i, j: (0, j)),
    ]
    if has_res:
        res = residual.astype(jnp.bfloat16)
        if res.shape[0] != mp:
            res = jnp.pad(res, ((0, mp - res.shape[0]), (0, 0)))
        inputs.append(res)
        in_specs.append(pl.BlockSpec((tm, tn), lambda i, j: (i, j)))

    no = np_ if (cout is None or np_ // tn > 1) else cout
    return pl.pallas_call(
        functools.partial(_mm_kernel, relu=relu, has_res=has_res),
        out_shape=jax.ShapeDtypeStruct((mp, no), jnp.bfloat16),
        grid=(mp // tm, np_ // tn),
        in_specs=in_specs,
        out_specs=pl.BlockSpec((tm, min(tn, no)), lambda i, j: (i, j)),
        compiler_params=pltpu.CompilerParams(
            dimension_semantics=("parallel", "parallel")),
    )(*inputs)


@functools.partial(jax.jit, static_argnames=("cout", "stride", "relu"))
def _conv1x1(x, p, residual=None, *, cout, stride=1, relu=True):
    if stride > 1:
        x = x[:, ::stride, ::stride, :]
    n, h, w, _ = x.shape
    a = x.reshape(n * h * w, -1)
    res = None if residual is None else residual.reshape(n * h * w, -1)
    out = _matmul_bn(a, p["w"], p["scale"], p["bias"], res, relu=relu,
                     cout=cout)
    return out[:n * h * w, :cout].reshape(n, h, w, cout)


@functools.partial(jax.jit,
                   static_argnames=("cout", "kh", "kw", "stride", "pad", "relu"))
def _conv_im2col(x, p, *, cout, kh, kw, stride, pad, relu):
    n, h, w, c = x.shape
    xp = jnp.pad(x, ((0, 0), (pad, pad), (pad, pad), (0, 0)))
    hp, wp = h + 2 * pad, w + 2 * pad
    ho = (hp - kh) // stride + 1
    wo = (wp - kw) // stride + 1
    cols = []
    for i in range(kh):
        for j in range(kw):
            cols.append(xp[:, i:i + stride * (ho - 1) + 1:stride,
                           j:j + stride * (wo - 1) + 1:stride, :])
    a = jnp.concatenate(cols, axis=-1).reshape(n * ho * wo, kh * kw * c)
    out = _matmul_bn(a, p["w"], p["scale"], p["bias"], relu=relu, cout=cout)
    return out[:n * ho * wo, :cout].reshape(n, ho, wo, cout)


def _c3_concat_kernel(x_ref, w_ref, s_ref, t_ref, o_ref, *, wp, tm):
    i = pl.program_id(1)
    halo = 2 * wp + 2
    base = pl.multiple_of(i * tm, 8)
    a_big = x_ref[pl.ds(base, tm + halo), :]
    taps = [a_big[dy * wp + dx:dy * wp + dx + tm, :]
            for dy in range(3) for dx in range(3)]
    a = jnp.concatenate(taps, axis=1)
    acc = jnp.dot(a, w_ref[...], preferred_element_type=jnp.float32)
    out = acc * s_ref[...] + t_ref[...]
    o_ref[...] = jnp.maximum(out[:, :o_ref.shape[1]], 0.0).astype(o_ref.dtype)


def _c3_taps_kernel(x_ref, w_ref, s_ref, t_ref, o_ref, *, wp, tm):
    i = pl.program_id(1)
    halo = 2 * wp + 2
    base = pl.multiple_of(i * tm, 8)
    a_big = x_ref[pl.ds(base, tm + halo), :]
    acc = None
    for dy in range(3):
        for dx in range(3):
            off = dy * wp + dx
            prod = jnp.dot(a_big[off:off + tm, :], w_ref[dy * 3 + dx],
                           preferred_element_type=jnp.float32)
            acc = prod if acc is None else acc + prod
    out = acc * s_ref[...] + t_ref[...]
    o_ref[...] = jnp.maximum(out[:, :o_ref.shape[1]], 0.0).astype(o_ref.dtype)


@functools.partial(jax.jit, static_argnames=("cout",))
def _conv3x3_fused(x, p, *, cout):
    n, h, w, cin = x.shape
    hp, wp = h + 2, w + 2
    m_img = hp * wp
    tm = 512 if m_img >= 512 else _ceil_to(m_img, 8)
    mp = _ceil_to(m_img, tm)
    np_ = p["w"].shape[2]
    tn = 256 if np_ % 256 == 0 else np_
    halo = 2 * wp + 2
    slab_rows = _ceil_to(mp + halo, 8)
    xp = jnp.pad(x, ((0, 0), (1, 1), (1, 1), (0, 0))).reshape(n, m_img, cin)
    slab = jnp.pad(xp, ((0, 0), (wp + 1, slab_rows - m_img - (wp + 1)), (0, 0)))

    wide = cin <= 128
    if wide:
        w2 = p["w"].reshape(9 * cin, np_)
        body = functools.partial(_c3_concat_kernel, wp=wp, tm=tm)
        w_spec = pl.BlockSpec((9 * cin, tn), lambda b, i, j: (0, j))
    else:
        w2 = p["w"]
        body = functools.partial(_c3_taps_kernel, wp=wp, tm=tm)
        w_spec = pl.BlockSpec((9, cin, tn), lambda b, i, j: (0, 0, j))

    no = np_ if np_ // tn > 1 else cout
    out = pl.pallas_call(
        body,
        out_shape=jax.ShapeDtypeStruct((n, mp, no), jnp.bfloat16),
        grid=(n, mp // tm, np_ // tn),
        in_specs=[
            pl.BlockSpec((None, slab_rows, cin), lambda b, i, j: (b, 0, 0)),
            w_spec,
            pl.BlockSpec((1, tn), lambda b, i, j: (0, j)),
            pl.BlockSpec((1, tn), lambda b, i, j: (0, j)),
        ],
        out_specs=pl.BlockSpec((None, tm, min(tn, no)), lambda b, i, j: (b, i, j)),
        compiler_params=pltpu.CompilerParams(
            dimension_semantics=("parallel", "parallel", "parallel")),
    )(slab, w2, p["scale"], p["bias"])
    out = out[:, :m_img, :cout].reshape(n, hp, wp, cout)
    return out[:, 1:1 + h, 1:1 + w, :]


def _stem_kernel(x_ref, w_ref, s_ref, t_ref, o_ref, *, wp, tm):
    i = pl.program_id(1)
    base = pl.multiple_of(i * tm, 8)
    a_big = x_ref[pl.ds(base, tm + 3 * wp + 3), :]
    acc = None
    for a in range(4):
        for c in range(4):
            off = a * wp + c
            prod = jnp.dot(a_big[off:off + tm, :], w_ref[a * 4 + c],
                           preferred_element_type=jnp.float32)
            acc = prod if acc is None else acc + prod
    out = acc * s_ref[...] + t_ref[...]
    o_ref[...] = out[:, :o_ref.shape[1]].astype(o_ref.dtype)


@jax.jit
def _stem_conv(x_nchw, p):
    n = x_nchw.shape[0]
    x = jnp.transpose(x_nchw, (0, 2, 3, 1)).astype(jnp.bfloat16)
    xp = jnp.pad(x, ((0, 0), (3, 3), (3, 3), (0, 0)))
    ph = xp.reshape(n, 115, 2, 115, 2, 3).transpose(0, 1, 3, 2, 4, 5)
    ph = ph.reshape(n, 115 * 115, 12)
    wp2 = 115
    m_img = 115 * 115
    tm = 512
    mp = _ceil_to(m_img, tm)
    slab_rows = _ceil_to(mp + 3 * wp2 + 3, 8)
    slab = jnp.pad(ph, ((0, 0), (0, slab_rows - m_img), (0, 0)))

    np_ = p["w"].shape[1]
    idx = []
    for a in range(4):
        for c in range(4):
            for b in range(2):
                for d in range(2):
                    for ch in range(3):
                        ky, kx = 2 * a + b, 2 * c + d
                        idx.append(ky * 21 + kx * 3 + ch if ky < 7 and kx < 7
                                   else 147)
    w4 = jnp.take(p["w"], jnp.asarray(idx), axis=0).reshape(16, 12, np_)

    out = pl.pallas_call(
        functools.partial(_stem_kernel, wp=wp2, tm=tm),
        out_shape=jax.ShapeDtypeStruct((n, mp, 64), jnp.bfloat16),
        grid=(n, mp // tm, 1),
        in_specs=[
            pl.BlockSpec((None, slab_rows, 12), lambda b, i, j: (b, 0, 0)),
            pl.BlockSpec((16, 12, np_), lambda b, i, j: (0, 0, 0)),
            pl.BlockSpec((1, np_), lambda b, i, j: (0, 0)),
            pl.BlockSpec((1, np_), lambda b, i, j: (0, 0)),
        ],
        out_specs=pl.BlockSpec((None, tm, 64), lambda b, i, j: (b, i, 0)),
        compiler_params=pltpu.CompilerParams(
            dimension_semantics=("parallel", "parallel", "parallel")),
    )(slab, w4, p["scale"], p["bias"])
    return out[:, :m_img, :].reshape(n, 115, 115, 64)


def _s2_kernel(x_ref, w_ref, s_ref, t_ref, o_ref, *, wp):
    acc = None
    for a in range(2):
        for c in range(2):
            off = a * wp + c
            prod = jnp.dot(x_ref[off:off + o_ref.shape[0], :], w_ref[a * 2 + c],
                           preferred_element_type=jnp.float32)
            acc = prod if acc is None else acc + prod
    out = acc * s_ref[...] + t_ref[...]
    o_ref[...] = jnp.maximum(out, 0.0).astype(o_ref.dtype)


@functools.partial(jax.jit, static_argnames=("cout",))
def _conv3x3_s2(x, p, *, cout):
    n, h, w, c = x.shape
    h2, w2 = (h + 2) // 2, (w + 2) // 2
    xp = jnp.pad(x, ((0, 0), (1, 1), (1, 1), (0, 0)))
    ph = xp.reshape(n, h2, 2, w2, 2, c).transpose(0, 1, 3, 2, 4, 5)
    ph = ph.reshape(n, h2 * w2, 4 * c)
    m_img = h2 * w2
    mp = _ceil_to(m_img, 8)
    slab_rows = _ceil_to(mp + w2 + 1, 8)
    slab = jnp.pad(ph, ((0, 0), (0, slab_rows - m_img), (0, 0)))

    kp, np_ = p["w"].shape
    wz = jnp.concatenate([p["w"], jnp.zeros((8, np_), p["w"].dtype)], axis=0)
    idx = []
    for a in range(2):
        for cc in range(2):
            for b in range(2):
                for d in range(2):
                    ky, kx = 2 * a + b, 2 * cc + d
                    for ch in range(c):
                        idx.append(ky * 3 * c + kx * c + ch
                                   if ky < 3 and kx < 3 else kp)
    w4 = jnp.take(wz, jnp.asarray(idx), axis=0).reshape(4, 4 * c, np_)

    tn = 256 if np_ % 256 == 0 else np_
    ho, wo = h // 2, w // 2
    out = pl.pallas_call(
        functools.partial(_s2_kernel, wp=w2),
        out_shape=jax.ShapeDtypeStruct((n, mp, np_), jnp.bfloat16),
        grid=(n, np_ // tn),
        in_specs=[
            pl.BlockSpec((None, slab_rows, 4 * c), lambda b, j: (b, 0, 0)),
            pl.BlockSpec((4, 4 * c, tn), lambda b, j: (0, 0, j)),
            pl.BlockSpec((1, tn), lambda b, j: (0, j)),
            pl.BlockSpec((1, tn), lambda b, j: (0, j)),
        ],
        out_specs=pl.BlockSpec((None, mp, tn), lambda b, j: (b, 0, j)),
        compiler_params=pltpu.CompilerParams(
            dimension_semantics=("parallel", "parallel")),
    )(slab, w4, p["scale"], p["bias"])
    out = out[:, :m_img, :cout].reshape(n, h2, w2, cout)
    return out[:, :ho, :wo, :]


_NEG = -1e30


def _pool_kernel(x_ref, o_ref, *, h, w, c):
    v = x_ref[0]
    vp = jnp.pad(v, ((1, 1), (1, 1), (0, 0)), constant_values=_NEG)
    hp, wp = h + 2, w + 2
    p = vp.reshape(hp, wp // 2, 2, c)
    ho, wo = h // 2, w // 2
    colmax = jnp.maximum(jnp.maximum(p[:, :wo, 0], p[:, :wo, 1]),
                         p[:, 1:wo + 1, 0])
    e = colmax.reshape(hp // 2, 2, wo, c)
    even, odd = e[:, 0], e[:, 1]
    out = jnp.maximum(jnp.maximum(even[:ho], odd[:ho]), even[1:ho + 1])
    o_ref[...] = out[None]


@jax.jit
def _maxpool_3x3_s2(x):
    n, h, w, c = x.shape
    return pl.pallas_call(
        functools.partial(_pool_kernel, h=h, w=w, c=c),
        out_shape=jax.ShapeDtypeStruct((n, h // 2, w // 2, c), x.dtype),
        grid=(n,),
        in_specs=[pl.BlockSpec((1, h, w, c), lambda b: (b, 0, 0, 0))],
        out_specs=pl.BlockSpec((1, h // 2, w // 2, c), lambda b: (b, 0, 0, 0)),
        compiler_params=pltpu.CompilerParams(dimension_semantics=("parallel",)),
    )(x)


def _gmax_kernel(x_ref, o_ref):
    o_ref[...] = jnp.max(x_ref[...].astype(jnp.float32), axis=0, keepdims=True)


@jax.jit
def _global_max(x):
    n, h, w, c = x.shape
    out = pl.pallas_call(
        _gmax_kernel,
        out_shape=jax.ShapeDtypeStruct((n, 1, c), jnp.float32),
        grid=(n,),
        in_specs=[pl.BlockSpec((None, h * w, c), lambda b: (b, 0, 0))],
        out_specs=pl.BlockSpec((None, 1, c), lambda b: (b, 0, 0)),
        compiler_params=pltpu.CompilerParams(dimension_semantics=("parallel",)),
    )(x.reshape(n, h * w, c))
    return out.reshape(n, c)


def _bottleneck(x, blk, planes, stride):
    out = _conv1x1(x, blk["c1"], cout=planes, relu=True)
    if stride == 1:
        out = _conv3x3_fused(out, blk["c2"], cout=planes)
    else:
        out = _conv3x3_s2(out, blk["c2"], cout=planes)
    if "ds" in blk:
        res = _conv1x1(x, blk["ds"], cout=planes * 4, stride=stride, relu=False)
    else:
        res = x
    return _conv1x1(out, blk["c3"], res, cout=planes * 4, relu=True)


def kernel(*args):
    it = iter(args)
    x = next(it)
    stem = {"w": next(it), "scale": next(it), "bias": next(it)}
    layers = []
    for planes, blocks, stride in _PLAN:
        stage = []
        for bi in range(blocks):
            blk = {}
            for nm in ("c1", "c2", "c3"):
                blk[nm] = {"w": next(it), "scale": next(it), "bias": next(it)}
            if bi == 0:
                blk["ds"] = {"w": next(it), "scale": next(it), "bias": next(it)}
            stage.append(blk)
        layers.append(stage)

    x = _stem_conv(x, stem)[:, :112, :112, :]
    x = _maxpool_3x3_s2(x)
    for (planes, blocks, stride), stage in zip(_PLAN, layers):
        for bi, blk in enumerate(stage):
            x = _bottleneck(x, blk, planes, stride if bi == 0 else 1)
    return _global_max(x)
